# baseline (speedup 1.0000x reference)
import jax
import jax.numpy as jnp
from jax import lax
from jax.experimental import pallas as pl
from jax.experimental.pallas import tpu as pltpu
from jax.experimental.pallas import tpu_sc as plsc

DIM = 768
NC = 2
NS = 16
NW = NC * NS
LC = 50
LQ = 20
K = 64


def _gather_body(idx_all, table, ctx_out, ctx_out2, q_out, idx_v, buf0, buf1,
                 sem0, sem1, ssem0, ssem1):
    wid = lax.axis_index("s") * NC + lax.axis_index("c")
    b = ctx_out.shape[1]
    kpb = b // K
    n_ctx_chunks = LC * kpb
    n_chunks = (LC + LQ) * kpb
    per_w = n_chunks // NW
    rows_w = per_w * K

    pltpu.sync_copy(idx_all.at[pl.ds(wid * rows_w, rows_w)], idx_v)

    def start_gather(j, buf, sem):
        pltpu.make_async_copy(
            table.at[idx_v.at[pl.ds(j * K, K)]], buf, sem).start()

    def wait_gather(buf, sem):
        pltpu.make_async_copy(
            table.at[idx_v.at[pl.ds(0, K)]], buf, sem).wait()

    def store_chunk(j, buf, ssem, ssem2):
        c = j * NW + wid

        @pl.when(c < n_ctx_chunks)
        def _():
            d1 = ctx_out.at[c // kpb, pl.ds((c % kpb) * K, K)]
            d2 = ctx_out2.at[c // kpb, pl.ds((c % kpb) * K, K)]
            cp1 = pltpu.make_async_copy(buf, d1, ssem)
            cp1.start()
            cp1.wait()
            pltpu.make_async_copy(d1, d2, ssem2).start()

        @pl.when(c >= n_ctx_chunks)
        def _():
            c2 = c - n_ctx_chunks
            pltpu.sync_copy(
                buf, q_out.at[c2 // kpb, pl.ds((c2 % kpb) * K, K)])

    start_gather(0, buf0, sem0)
    start_gather(1, buf1, sem1)

    @pl.loop(0, per_w // 2)
    def _pair(i):
        for buf, sem, par in ((buf0, sem0, 0), (buf1, sem1, 1)):
            j = 2 * i + par
            wait_gather(buf, sem)
            store_chunk(j, buf, ssem0, ssem1)

            @pl.when(j + 2 < per_w)
            def _():
                start_gather(j + 2, buf, sem)

    if per_w % 2:
        wait_gather(buf0, sem0)
        store_chunk(per_w - 1, buf0, ssem0, ssem1)

    @pl.loop(0, n_ctx_chunks // NW)
    def _drain(i):
        pltpu.make_async_copy(
            ctx_out.at[0, pl.ds(0, K)], ctx_out2.at[0, pl.ds(0, K)],
            ssem1).wait()


@jax.jit
def _gather(idx_all, table):
    b = 1024
    mesh = plsc.VectorSubcoreMesh(core_axis_name="c", subcore_axis_name="s")
    f = pl.kernel(
        _gather_body,
        out_type=(
            jax.ShapeDtypeStruct((LC, b, DIM), jnp.float32),
            jax.ShapeDtypeStruct((LC, b, DIM), jnp.float32),
            jax.ShapeDtypeStruct((LQ, b, DIM), jnp.float32),
        ),
        mesh=mesh,
        scratch_types=[
            pltpu.VMEM((idx_all.shape[0] // NW,), jnp.int32),
            pltpu.VMEM((K, DIM), jnp.float32),
            pltpu.VMEM((K, DIM), jnp.float32),
            pltpu.SemaphoreType.DMA,
            pltpu.SemaphoreType.DMA,
            pltpu.SemaphoreType.DMA,
            pltpu.SemaphoreType.DMA,
        ],
    )
    return f(idx_all, table)


def kernel(context, context_lengths, question, question_lengths, table):
    idx_flat = jnp.concatenate(
        [context.T.reshape(-1), question.T.reshape(-1)]).astype(jnp.int32)
    n_chunks = idx_flat.shape[0] // K
    idx_all = jnp.transpose(
        idx_flat.reshape(n_chunks // NW, NW, K), (1, 0, 2)).reshape(-1)
    ctx_t, ctx_t2, q_t = _gather(idx_all, table)
    ctx_e = jnp.transpose(ctx_t, (1, 0, 2))
    ctx_e2 = jnp.transpose(ctx_t2, (1, 0, 2))
    q_e = jnp.transpose(q_t, (1, 0, 2))
    return (ctx_e, ctx_e2, q_e)
